# Initial kernel scaffold; baseline (speedup 1.0000x reference)
import functools

import jax
import jax.numpy as jnp
from jax import lax
from jax.experimental import pallas as pl
from jax.experimental.pallas import tpu as pltpu
from jax.experimental.pallas import tpu_sc as plsc

N = 10000
D = 128
T = 4
CH = 128
NW = 32
NSUB = 16
NCORE = 2
EPAD_UNIT = NW * CH
NP = ((N // NSUB) + 1) * NSUB
RB = 1000
HI = lax.Precision.HIGHEST

_mesh = plsc.VectorSubcoreMesh(core_axis_name="c", subcore_axis_name="s")


def _make_deg_kernel(nchunk):
    @functools.partial(
        pl.kernel,
        mesh=_mesh,
        out_type=jax.ShapeDtypeStruct((NCORE, NP, 16), jnp.float32),
        scratch_types=[
            pltpu.VMEM((CH,), jnp.int32),
            pltpu.VMEM((CH, 16), jnp.float32),
            pltpu.VMEM_SHARED((NP, 16), jnp.float32),
        ],
    )
    def degk(dstr, ones_hbm, zeros_hbm, out, dst_v, ones_v, accd):
        c = lax.axis_index("c")
        s = lax.axis_index("s")
        wid = s * NCORE + c
        rows = NP // NSUB
        base = s * rows
        pltpu.sync_copy(ones_hbm, ones_v)
        pltpu.sync_copy(zeros_hbm.at[pl.ds(base, rows)], accd.at[pl.ds(base, rows)])
        plsc.subcore_barrier()

        def body(j, _):
            pltpu.sync_copy(dstr.at[wid * nchunk + j], dst_v)
            pltpu.sync_copy(ones_v, accd.at[dst_v], add=True)
            return ()

        lax.fori_loop(0, nchunk, body, ())
        plsc.subcore_barrier()
        for cc in range(NCORE):
            @pl.when(c == cc)
            def _():
                pltpu.sync_copy(accd.at[pl.ds(base, rows)],
                                out.at[cc, pl.ds(base, rows)])

    return degk


def _make_agg_kernel(tt, nchunk):
    @functools.partial(
        pl.kernel,
        mesh=_mesh,
        out_type=jax.ShapeDtypeStruct((tt, NCORE, NP, D), jnp.float32),
        scratch_types=[
            pltpu.VMEM((CH,), jnp.int32),
            pltpu.VMEM((CH,), jnp.int32),
            pltpu.VMEM((CH, D), jnp.float32),
            pltpu.VMEM_SHARED((NP, D), jnp.float32),
            pltpu.SemaphoreType.DMA,
        ],
    )
    def aggk(*refs):
        hhs = refs[:tt]
        srcr, dstr, zeros_hbm, out = refs[tt:tt + 4]
        src_v, dst_v, rows_v, acc, sem = refs[tt + 4:]
        c = lax.axis_index("c")
        s = lax.axis_index("s")
        wid = s * NCORE + c
        rows = NP // NSUB
        base = s * rows
        for t in range(tt):
            pltpu.sync_copy(zeros_hbm.at[pl.ds(base, rows)],
                            acc.at[pl.ds(base, rows)])
            plsc.subcore_barrier()

            def body(j, _):
                pltpu.sync_copy(srcr.at[wid * nchunk + j], src_v)
                pltpu.sync_copy(dstr.at[wid * nchunk + j], dst_v)
                pltpu.async_copy(hhs[t].at[src_v], rows_v, sem).wait()
                pltpu.sync_copy(rows_v, acc.at[dst_v], add=True)
                return ()

            lax.fori_loop(0, nchunk, body, ())
            plsc.subcore_barrier()
            for cc in range(NCORE):
                @pl.when(c == cc)
                def _():
                    pltpu.sync_copy(acc.at[pl.ds(base, rows)],
                                    out.at[t, cc, pl.ds(base, rows)])
            plsc.subcore_barrier()

    return aggk


def _dis_idiv(degp):
    deg = degp[0, :, 0:1] + degp[1, :, 0:1] + 1.0
    dis = lax.rsqrt(deg)
    return dis, 1.0 / deg


def _prep_body(x_ref, degp_ref, hx_ref):
    dis, _ = _dis_idiv(degp_ref[...])
    hx_ref[...] = dis * x_ref[...]


def _left_body(x_ref, degp_ref, p_ref, W1_ref, b1_ref, nW1_ref, nb1_ref,
               s1_ref, hs_ref):
    x = x_ref[...]
    dis, idiv = _dis_idiv(degp_ref[...])
    agg = p_ref[0] + p_ref[1]
    el = jnp.dot(dis * agg + idiv * x, W1_ref[...],
                 preferred_element_type=jnp.float32, precision=HI) + b1_ref[...]
    mean = jnp.mean(el, axis=1, keepdims=True)
    ctr = el - mean
    S2 = jnp.sum(ctr * ctr, axis=1, keepdims=True)
    P = jnp.dot(ctr, nW1_ref[...],
                preferred_element_type=jnp.float32, precision=HI)
    nb1 = nb1_ref[...]
    v = jnp.zeros_like(P)
    for t in range(1, T + 1):
        std = jnp.sqrt(t * S2 / (t * D - 1))
        nl = P / (std + 1e-8) + nb1
        v = v + (nl - v) * 0.5
        s1 = (v - 1.0 >= 0.0).astype(jnp.float32)
        v = v * (1.0 - s1)
        s1_ref[t - 1] = s1
        hs_ref[t - 1] = dis * s1


def _right_body(q_ref, degp_ref, s1_ref, W2_ref, b2_ref, nW2_ref, nb2_ref,
                out_ref):
    dis, idiv = _dis_idiv(degp_ref[...])
    W2 = W2_ref[...]
    b2 = b2_ref[...]
    nW2 = nW2_ref[...]
    nb2 = nb2_ref[...]
    colsum2 = jnp.sum(nW2, axis=0, keepdims=True)
    ers = []
    for t in range(T):
        aggt = q_ref[t, 0] + q_ref[t, 1]
        ers.append(jnp.dot(dis * aggt + idiv * s1_ref[t], W2,
                           preferred_element_type=jnp.float32, precision=HI) + b2)
    v2 = jnp.zeros_like(ers[0])
    acc = jnp.zeros_like(ers[0])
    run_sum = jnp.zeros_like(ers[0][:, 0:1])
    for t in range(1, T + 1):
        run_sum = run_sum + jnp.sum(ers[t - 1], axis=1, keepdims=True)
        mean_t = run_sum / (t * D)
        S = jnp.zeros_like(run_sum)
        for tau in range(t):
            dlt = ers[tau] - mean_t
            S = S + jnp.sum(dlt * dlt, axis=1, keepdims=True)
        std = jnp.sqrt(S / (t * D - 1))
        nr = (jnp.dot(ers[t - 1], nW2, preferred_element_type=jnp.float32,
                      precision=HI) - mean_t * colsum2) / (std + 1e-8) + nb2
        v2 = v2 + (nr - v2) * 0.5
        s2 = (v2 - 1.0 >= 0.0).astype(jnp.float32)
        v2 = v2 * (1.0 - s2)
        acc = acc + s2
    out_ref[...] = acc * (1.0 / T)


def kernel(x, edge_index, W1, b1, W2, b2, nW1, nb1, nW2, nb2):
    src = edge_index[0]
    dst = edge_index[1]
    e = src.shape[0]
    epad = ((e + EPAD_UNIT - 1) // EPAD_UNIT) * EPAD_UNIT
    nchunk = epad // EPAD_UNIT
    pad = epad - e
    srcp = jnp.concatenate([src, jnp.zeros((pad,), jnp.int32)]).reshape(-1, CH)
    dstp = jnp.concatenate([dst, jnp.full((pad,), N, jnp.int32)]).reshape(-1, CH)

    zeros_d = jnp.zeros((NP, D), jnp.float32)
    zeros_16 = jnp.zeros((NP, 16), jnp.float32)
    ones_16 = jnp.ones((CH, 16), jnp.float32)

    degp = _make_deg_kernel(nchunk)(dstp, ones_16, zeros_16)
    degp_n = degp[:, :N, :]

    grid = (N // RB,)
    hx = pl.pallas_call(
        _prep_body,
        grid=grid,
        in_specs=[
            pl.BlockSpec((RB, D), lambda i: (i, 0)),
            pl.BlockSpec((NCORE, RB, 16), lambda i: (0, i, 0)),
        ],
        out_specs=pl.BlockSpec((RB, D), lambda i: (i, 0)),
        out_shape=jax.ShapeDtypeStruct((N, D), jnp.float32),
    )(x, degp_n)

    hxp = jnp.concatenate([hx, jnp.zeros((NP - N, D), jnp.float32)], axis=0)
    p = _make_agg_kernel(1, nchunk)(hxp, srcp, dstp, zeros_d)
    p_n = p[0, :, :N, :]

    b1r = b1.reshape(1, D)
    nb1r = nb1.reshape(1, D)
    s1_all, hs_all = pl.pallas_call(
        _left_body,
        grid=grid,
        in_specs=[
            pl.BlockSpec((RB, D), lambda i: (i, 0)),
            pl.BlockSpec((NCORE, RB, 16), lambda i: (0, i, 0)),
            pl.BlockSpec((NCORE, RB, D), lambda i: (0, i, 0)),
            pl.BlockSpec((D, D), lambda i: (0, 0)),
            pl.BlockSpec((1, D), lambda i: (0, 0)),
            pl.BlockSpec((D, D), lambda i: (0, 0)),
            pl.BlockSpec((1, D), lambda i: (0, 0)),
        ],
        out_specs=[
            pl.BlockSpec((T, RB, D), lambda i: (0, i, 0)),
            pl.BlockSpec((T, RB, D), lambda i: (0, i, 0)),
        ],
        out_shape=[
            jax.ShapeDtypeStruct((T, N, D), jnp.float32),
            jax.ShapeDtypeStruct((T, N, D), jnp.float32),
        ],
    )(x, degp_n, p_n, W1, b1r, nW1, nb1r)

    zpad = jnp.zeros((NP - N, D), jnp.float32)
    hs_args = [jnp.concatenate([hs_all[t], zpad], axis=0) for t in range(T)]
    q = _make_agg_kernel(T, nchunk)(*hs_args, srcp, dstp, zeros_d)
    q_n = q[:, :, :N, :]

    b2r = b2.reshape(1, D)
    nb2r = nb2.reshape(1, D)
    out = pl.pallas_call(
        _right_body,
        grid=grid,
        in_specs=[
            pl.BlockSpec((T, NCORE, RB, D), lambda i: (0, 0, i, 0)),
            pl.BlockSpec((NCORE, RB, 16), lambda i: (0, i, 0)),
            pl.BlockSpec((T, RB, D), lambda i: (0, i, 0)),
            pl.BlockSpec((D, D), lambda i: (0, 0)),
            pl.BlockSpec((1, D), lambda i: (0, 0)),
            pl.BlockSpec((D, D), lambda i: (0, 0)),
            pl.BlockSpec((1, D), lambda i: (0, 0)),
        ],
        out_specs=pl.BlockSpec((RB, D), lambda i: (i, 0)),
        out_shape=jax.ShapeDtypeStruct((N, D), jnp.float32),
    )(q_n, degp_n, s1_all, W2, b2r, nW2, nb2r)
    return out


# speedup vs baseline: 7.3500x; 7.3500x over previous
import functools

import jax
import jax.numpy as jnp
from jax import lax
from jax.experimental import pallas as pl
from jax.experimental.pallas import tpu as pltpu
from jax.experimental.pallas import tpu_sc as plsc

N = 10000
D = 128
T = 4
CH = 128
NW = 32
NSUB = 16
NCORE = 2
EPAD_UNIT = NW * CH
NP = ((N + 1 + 127) // 128) * 128
RB = 1000

_mesh = plsc.VectorSubcoreMesh(core_axis_name="c", subcore_axis_name="s")


def _make_deg_kernel(nchunk):
    @functools.partial(
        pl.kernel,
        mesh=_mesh,
        out_type=jax.ShapeDtypeStruct((NCORE, NP, D), jnp.float32),
        scratch_types=[
            pltpu.VMEM((CH,), jnp.int32),
            pltpu.VMEM((CH, D), jnp.float32),
            pltpu.VMEM_SHARED((NP, D), jnp.float32),
        ],
    )
    def degk(dstr, ones_hbm, zeros_hbm, out, dst_v, ones_v, accd):
        c = lax.axis_index("c")
        s = lax.axis_index("s")
        wid = s * NCORE + c
        rows = NP // NSUB
        base = s * rows
        pltpu.sync_copy(ones_hbm, ones_v)
        pltpu.sync_copy(zeros_hbm.at[pl.ds(base, rows)], accd.at[pl.ds(base, rows)])
        plsc.subcore_barrier()

        def body(j, _):
            pltpu.sync_copy(dstr.at[wid * nchunk + j], dst_v)
            pltpu.sync_copy(ones_v, accd.at[dst_v], add=True)
            return ()

        lax.fori_loop(0, nchunk, body, ())
        plsc.subcore_barrier()
        for cc in range(NCORE):
            @pl.when(c == cc)
            def _():
                pltpu.sync_copy(accd.at[pl.ds(base, rows)],
                                out.at[cc, pl.ds(base, rows)])

    return degk


def _make_agg_kernel(tt, nchunk):
    @functools.partial(
        pl.kernel,
        mesh=_mesh,
        out_type=jax.ShapeDtypeStruct((tt, NCORE, NP, D), jnp.float32),
        scratch_types=[
            pltpu.VMEM((CH,), jnp.int32),
            pltpu.VMEM((CH,), jnp.int32),
            pltpu.VMEM((CH, D), jnp.float32),
            pltpu.VMEM_SHARED((NP, D), jnp.float32),
            pltpu.SemaphoreType.DMA,
        ],
    )
    def aggk(*refs):
        hhs = refs[:tt]
        srcr, dstr, zeros_hbm, out = refs[tt:tt + 4]
        src_v, dst_v, rows_v, acc, sem = refs[tt + 4:]
        c = lax.axis_index("c")
        s = lax.axis_index("s")
        wid = s * NCORE + c
        rows = NP // NSUB
        base = s * rows
        for t in range(tt):
            pltpu.sync_copy(zeros_hbm.at[pl.ds(base, rows)],
                            acc.at[pl.ds(base, rows)])
            plsc.subcore_barrier()

            def body(j, _):
                pltpu.sync_copy(srcr.at[wid * nchunk + j], src_v)
                pltpu.sync_copy(dstr.at[wid * nchunk + j], dst_v)
                pltpu.async_copy(hhs[t].at[src_v], rows_v, sem).wait()
                pltpu.sync_copy(rows_v, acc.at[dst_v], add=True)
                return ()

            lax.fori_loop(0, nchunk, body, ())
            plsc.subcore_barrier()
            for cc in range(NCORE):
                @pl.when(c == cc)
                def _():
                    pltpu.sync_copy(acc.at[pl.ds(base, rows)],
                                    out.at[t, cc, pl.ds(base, rows)])
            plsc.subcore_barrier()

    return aggk


def _dis_idiv(degp):
    deg = degp[0, :, 0:1] + degp[1, :, 0:1] + 1.0
    dis = 1.0 / jnp.sqrt(deg)
    return dis, dis * dis


def _prep_body(x_ref, degp_ref, W1_ref, xw_ref, hxw_ref):
    dis, _ = _dis_idiv(degp_ref[...])
    xw = jnp.dot(x_ref[...], W1_ref[...], preferred_element_type=jnp.float32)
    xw_ref[...] = xw
    hxw_ref[...] = dis * xw


def _left_body(xw_ref, degp_ref, p_ref, b1_ref, nW1_ref, nb1_ref, W2_ref,
               sw_ref, hs2_ref):
    xw = xw_ref[...]
    dis, idiv = _dis_idiv(degp_ref[...])
    el = dis * (p_ref[0] + p_ref[1]) + idiv * xw + b1_ref[...]
    mean = jnp.mean(el, axis=1, keepdims=True)
    ctr = el - mean
    S2 = jnp.sum(ctr * ctr, axis=1, keepdims=True)
    nb1 = nb1_ref[...]
    nW1 = nW1_ref[...]
    W2 = W2_ref[...]
    v = jnp.zeros_like(el)
    for t in range(1, T + 1):
        std = jnp.sqrt(t * S2 / (t * D - 1))
        nl = jnp.dot(ctr / (std + 1e-8), nW1,
                     preferred_element_type=jnp.float32) + nb1
        v = v + (nl - v) * 0.5
        s1 = (v - 1.0 >= 0.0).astype(jnp.float32)
        v = v * (1.0 - s1)
        sw = jnp.dot(s1, W2, preferred_element_type=jnp.float32)
        sw_ref[t - 1] = sw
        hs2_ref[t - 1] = dis * sw


def _right_body(q_ref, degp_ref, sw_ref, b2_ref, nW2_ref, nb2_ref, out_ref):
    dis, idiv = _dis_idiv(degp_ref[...])
    b2 = b2_ref[...]
    nW2 = nW2_ref[...]
    nb2 = nb2_ref[...]
    ers = []
    for t in range(T):
        ers.append(dis * (q_ref[t, 0] + q_ref[t, 1]) + idiv * sw_ref[t] + b2)
    v2 = jnp.zeros_like(ers[0])
    acc = jnp.zeros_like(ers[0])
    run_sum = jnp.zeros_like(ers[0][:, 0:1])
    for t in range(1, T + 1):
        run_sum = run_sum + jnp.sum(ers[t - 1], axis=1, keepdims=True)
        mean_t = run_sum / (t * D)
        S = jnp.zeros_like(run_sum)
        for tau in range(t):
            dlt = ers[tau] - mean_t
            S = S + jnp.sum(dlt * dlt, axis=1, keepdims=True)
        std = jnp.sqrt(S / (t * D - 1))
        nr = jnp.dot((ers[t - 1] - mean_t) / (std + 1e-8), nW2,
                     preferred_element_type=jnp.float32) + nb2
        v2 = v2 + (nr - v2) * 0.5
        s2 = (v2 - 1.0 >= 0.0).astype(jnp.float32)
        v2 = v2 * (1.0 - s2)
        acc = acc + s2
    out_ref[...] = acc * (1.0 / T)


def kernel(x, edge_index, W1, b1, W2, b2, nW1, nb1, nW2, nb2):
    src = edge_index[0]
    dst = edge_index[1]
    e = src.shape[0]
    epad = ((e + EPAD_UNIT - 1) // EPAD_UNIT) * EPAD_UNIT
    nchunk = epad // EPAD_UNIT
    pad = epad - e
    srcp = jnp.concatenate([src, jnp.zeros((pad,), jnp.int32)]).reshape(-1, CH)
    dstp = jnp.concatenate([dst, jnp.full((pad,), N, jnp.int32)]).reshape(-1, CH)

    zeros_d = jnp.zeros((NP, D), jnp.float32)
    ones_d = jnp.ones((CH, D), jnp.float32)

    degp = _make_deg_kernel(nchunk)(dstp, ones_d, zeros_d)
    degp_n = degp[:, :N, :16]

    grid = (N // RB,)
    xw, hxw = pl.pallas_call(
        _prep_body,
        grid=grid,
        in_specs=[
            pl.BlockSpec((RB, D), lambda i: (i, 0)),
            pl.BlockSpec((NCORE, RB, 16), lambda i: (0, i, 0)),
            pl.BlockSpec((D, D), lambda i: (0, 0)),
        ],
        out_specs=[
            pl.BlockSpec((RB, D), lambda i: (i, 0)),
            pl.BlockSpec((RB, D), lambda i: (i, 0)),
        ],
        out_shape=[
            jax.ShapeDtypeStruct((N, D), jnp.float32),
            jax.ShapeDtypeStruct((N, D), jnp.float32),
        ],
    )(x, degp_n, W1)

    hxwp = jnp.concatenate([hxw, jnp.zeros((NP - N, D), jnp.float32)], axis=0)
    p = _make_agg_kernel(1, nchunk)(hxwp, srcp, dstp, zeros_d)
    p_n = p[0, :, :N, :]

    b1r = b1.reshape(1, D)
    nb1r = nb1.reshape(1, D)
    sw_all, hs2_all = pl.pallas_call(
        _left_body,
        grid=grid,
        in_specs=[
            pl.BlockSpec((RB, D), lambda i: (i, 0)),
            pl.BlockSpec((NCORE, RB, 16), lambda i: (0, i, 0)),
            pl.BlockSpec((NCORE, RB, D), lambda i: (0, i, 0)),
            pl.BlockSpec((1, D), lambda i: (0, 0)),
            pl.BlockSpec((D, D), lambda i: (0, 0)),
            pl.BlockSpec((1, D), lambda i: (0, 0)),
            pl.BlockSpec((D, D), lambda i: (0, 0)),
        ],
        out_specs=[
            pl.BlockSpec((T, RB, D), lambda i: (0, i, 0)),
            pl.BlockSpec((T, RB, D), lambda i: (0, i, 0)),
        ],
        out_shape=[
            jax.ShapeDtypeStruct((T, N, D), jnp.float32),
            jax.ShapeDtypeStruct((T, N, D), jnp.float32),
        ],
    )(xw, degp_n, p_n, b1r, nW1, nb1r, W2)

    zpad = jnp.zeros((NP - N, D), jnp.float32)
    hs_args = [jnp.concatenate([hs2_all[t], zpad], axis=0) for t in range(T)]
    q = _make_agg_kernel(T, nchunk)(*hs_args, srcp, dstp, zeros_d)
    q_n = q[:, :, :N, :]

    b2r = b2.reshape(1, D)
    nb2r = nb2.reshape(1, D)
    out = pl.pallas_call(
        _right_body,
        grid=grid,
        in_specs=[
            pl.BlockSpec((T, NCORE, RB, D), lambda i: (0, 0, i, 0)),
            pl.BlockSpec((NCORE, RB, 16), lambda i: (0, i, 0)),
            pl.BlockSpec((T, RB, D), lambda i: (0, i, 0)),
            pl.BlockSpec((1, D), lambda i: (0, 0)),
            pl.BlockSpec((D, D), lambda i: (0, 0)),
            pl.BlockSpec((1, D), lambda i: (0, 0)),
        ],
        out_specs=pl.BlockSpec((RB, D), lambda i: (i, 0)),
        out_shape=jax.ShapeDtypeStruct((N, D), jnp.float32),
    )(q_n, degp_n, sw_all, b2r, nW2, nb2r)
    return out
